# baseline (speedup 1.0000x reference)
import jax
import jax.numpy as jnp
from jax import lax
from jax.experimental import pallas as pl
from jax.experimental.pallas import tpu as pltpu

_THRESHOLD = 0.35
_NEGPOS_RATIO = 7
_VAR0, _VAR1 = 0.1, 0.2
_B = 32
_P = 16800
_NOBJ = 32
_ROWS = 8
_COLS = 2176
_PPAD = _ROWS * _COLS

_LOC, _CONF, _BLUR, _EXPR, _ILLU, _OCCL, _POSE = 0, 4, 6, 9, 11, 13, 16


def _smooth_l1(x):
    ax = jnp.abs(x)
    return jnp.where(ax < 1.0, 0.5 * x * x, ax - 0.5)


def _lse2(a, b):
    m = jnp.maximum(a, b)
    return m + jnp.log(jnp.exp(a - m) + jnp.exp(b - m))


def _lse3(a, b, c):
    m = jnp.maximum(jnp.maximum(a, b), c)
    return m + jnp.log(jnp.exp(a - m) + jnp.exp(b - m) + jnp.exp(c - m))


def _match_body(tg_ref, priors_ref, data_ref,
                out_lcp, out_l, out_cpos, out_b, out_e, out_i, out_o, out_p,
                out_np):
    f32 = jnp.float32
    bimg = pl.program_id(0)

    pcx = priors_ref[0]
    pcy = priors_ref[1]
    pw = priors_ref[2]
    ph = priors_ref[3]
    pxmin = pcx - pw / 2.0
    pymin = pcy - ph / 2.0
    pxmax = pcx + pw / 2.0
    pymax = pcy + ph / 2.0
    area_b = (pxmax - pxmin) * (pymax - pymin)

    riota = lax.broadcasted_iota(jnp.int32, (_ROWS, _COLS), 0)
    ciota = lax.broadcasted_iota(jnp.int32, (_ROWS, _COLS), 1)
    piota = riota * _COLS + ciota
    valid = piota < _P

    best_ov = jnp.full((_ROWS, _COLS), -1.0, f32)
    best_j = jnp.zeros((_ROWS, _COLS), jnp.int32)
    bp_list = []
    for j in range(_NOBJ):
        x1 = tg_ref[bimg, j, 0]
        y1 = tg_ref[bimg, j, 1]
        x2 = tg_ref[bimg, j, 2]
        y2 = tg_ref[bimg, j, 3]
        iw = jnp.maximum(jnp.minimum(x2, pxmax) - jnp.maximum(x1, pxmin), 0.0)
        ih = jnp.maximum(jnp.minimum(y2, pymax) - jnp.maximum(y1, pymin), 0.0)
        inter = iw * ih
        area_a = (x2 - x1) * (y2 - y1)
        ov = inter / (area_a + area_b - inter)
        upd = ov > best_ov
        best_ov = jnp.where(upd, ov, best_ov)
        best_j = jnp.where(upd, j, best_j)
        mx = jnp.max(ov)
        bp_list.append(jnp.min(jnp.where(ov == mx, piota, _PPAD)))
    for j in range(_NOBJ):
        m = piota == bp_list[j]
        best_ov = jnp.where(m, 2.0, best_ov)
        best_j = jnp.where(m, j, best_j)

    pos = jnp.logical_and(best_ov >= _THRESHOLD, valid)

    def gather_chan(cidx):
        acc = jnp.zeros((_ROWS, _COLS), f32)
        for j in range(_NOBJ):
            acc = jnp.where(best_j == j, tg_ref[bimg, j, cidx], acc)
        return acc

    mx1 = gather_chan(0)
    my1 = gather_chan(1)
    mx2 = gather_chan(2)
    my2 = gather_chan(3)
    ci = gather_chan(4).astype(jnp.int32)
    abl = ci & 3
    aex = (ci >> 2) & 3
    ail = (ci >> 4) & 3
    aoc = (ci >> 6) & 3
    apo = (ci >> 8) & 3

    g_cx = ((mx1 + mx2) / 2.0 - pcx) / (_VAR0 * pw)
    g_cy = ((my1 + my2) / 2.0 - pcy) / (_VAR0 * ph)
    g_w = jnp.log((mx2 - mx1) / pw) / _VAR1
    g_h = jnp.log((my2 - my1) / ph) / _VAR1
    d = data_ref[0]
    sl = (_smooth_l1(d[_LOC + 0] - g_cx) + _smooth_l1(d[_LOC + 1] - g_cy)
          + _smooth_l1(d[_LOC + 2] - g_w) + _smooth_l1(d[_LOC + 3] - g_h))
    loss_l = jnp.sum(jnp.where(pos, sl, 0.0))

    def ce2(off, att):
        h0 = d[off]
        h1 = d[off + 1]
        picked = jnp.where(att == 0, h0, h1)
        return _lse2(h0, h1) - picked

    def ce3(off, att):
        h0 = d[off]
        h1 = d[off + 1]
        h2 = d[off + 2]
        picked = jnp.where(att == 0, h0, jnp.where(att == 1, h1, h2))
        return _lse3(h0, h1, h2) - picked

    loss_b = jnp.sum(jnp.where(pos, ce3(_BLUR, abl), 0.0))
    loss_e = jnp.sum(jnp.where(pos, ce2(_EXPR, aex), 0.0))
    loss_i = jnp.sum(jnp.where(pos, ce2(_ILLU, ail), 0.0))
    loss_o = jnp.sum(jnp.where(pos, ce3(_OCCL, aoc), 0.0))
    loss_p = jnp.sum(jnp.where(pos, ce2(_POSE, apo), 0.0))

    c0 = d[_CONF]
    c1 = d[_CONF + 1]
    lse_c = _lse2(c0, c1)
    loss_c_pos = jnp.sum(jnp.where(pos, lse_c - c1, 0.0))
    lcp = jnp.where(pos, 0.0, jnp.where(valid, lse_c - c0, 0.0))
    out_lcp[0] = lcp

    num_pos_i = jnp.sum(jnp.where(pos, 1, 0))
    out_l[0, 0, 0] = loss_l
    out_cpos[0, 0, 0] = loss_c_pos
    out_b[0, 0, 0] = loss_b
    out_e[0, 0, 0] = loss_e
    out_i[0, 0, 0] = loss_i
    out_o[0, 0, 0] = loss_o
    out_p[0, 0, 0] = loss_p
    out_np[0, 0, 0] = num_pos_i.astype(f32)


def _mine_body(lcp_ref, kv_ref, out_neg):
    f32 = jnp.float32
    lcp = lcp_ref[...]
    bits = lax.bitcast_convert_type(lcp, jnp.int32)
    k = kv_ref[...]
    lo = jnp.zeros((_B, 1, 1), jnp.int32)
    hi = jnp.full((_B, 1, 1), 2147483647, jnp.int32)
    for _ in range(31):
        mid = lo + ((hi - lo) >> 1)
        cnt = jnp.sum(jnp.where(bits >= mid, 1, 0), axis=(1, 2), keepdims=True)
        ge_k = cnt >= k
        lo = jnp.where(ge_k, mid, lo)
        hi = jnp.where(ge_k, hi, mid)
    t = lax.bitcast_convert_type(lo, f32)
    gt = lcp > t
    cnt_gt = jnp.sum(jnp.where(gt, 1.0, 0.0), axis=(1, 2), keepdims=True)
    sum_gt = jnp.sum(jnp.where(gt, lcp, 0.0), axis=(1, 2), keepdims=True)
    neg_rows = sum_gt + (k.astype(f32) - cnt_gt) * t
    out_neg[0, 0] = jnp.sum(neg_rows)


def kernel(loc_data, conf_data, blur_data, expression_data, illumination_data,
           occlusion_data, pose_data, priors, targets):
    data = jnp.concatenate([loc_data, conf_data, blur_data, expression_data,
                            illumination_data, occlusion_data, pose_data],
                           axis=-1)
    dataT = jnp.transpose(data, (0, 2, 1))
    dataT = jnp.pad(dataT, ((0, 0), (0, 0), (0, _PPAD - _P)))
    dataT = dataT.reshape(_B, 18, _ROWS, _COLS)
    priorsT = jnp.pad(priors.T, ((0, 0), (0, _PPAD - _P))).reshape(
        4, _ROWS, _COLS)
    code = (targets[..., 4] + 4.0 * targets[..., 5] + 16.0 * targets[..., 6]
            + 64.0 * targets[..., 7] + 256.0 * targets[..., 8])
    tg2 = jnp.stack([targets[..., 0], targets[..., 1], targets[..., 2],
                     targets[..., 3], code], axis=-1)

    outs = pl.pallas_call(
        _match_body,
        grid=(_B,),
        in_specs=[
            pl.BlockSpec(memory_space=pltpu.SMEM),
            pl.BlockSpec((4, _ROWS, _COLS), lambda b: (0, 0, 0)),
            pl.BlockSpec((1, 18, _ROWS, _COLS), lambda b: (b, 0, 0, 0)),
        ],
        out_specs=[pl.BlockSpec((1, _ROWS, _COLS), lambda b: (b, 0, 0))] +
                  [pl.BlockSpec((1, 1, 1), lambda b: (b, 0, 0),
                                memory_space=pltpu.SMEM)] * 8,
        out_shape=[jax.ShapeDtypeStruct((_B, _ROWS, _COLS), jnp.float32)] +
                  [jax.ShapeDtypeStruct((_B, 1, 1), jnp.float32)] * 8,
    )(tg2, priorsT, dataT)

    (lcp_all, o_l, o_cpos, o_b, o_e, o_i, o_o, o_p, o_np) = outs
    kv = jnp.minimum(_NEGPOS_RATIO * o_np.astype(jnp.int32), _P - 1)

    neg_sum = pl.pallas_call(
        _mine_body,
        in_specs=[pl.BlockSpec(memory_space=pltpu.VMEM),
                  pl.BlockSpec(memory_space=pltpu.VMEM)],
        out_specs=pl.BlockSpec(memory_space=pltpu.SMEM),
        out_shape=jax.ShapeDtypeStruct((1, 1), jnp.float32),
    )(lcp_all, kv)

    s_l = jnp.sum(o_l)
    s_cpos = jnp.sum(o_cpos)
    s_b = jnp.sum(o_b)
    s_e = jnp.sum(o_e)
    s_i = jnp.sum(o_i)
    s_o = jnp.sum(o_o)
    s_p = jnp.sum(o_p)
    n = jnp.maximum(jnp.sum(o_np), 1.0)
    loss_c = s_cpos + neg_sum[0, 0]
    return (s_l / n, loss_c / n, s_b / n, s_e / n, s_i / n, s_o / n, s_p / n)


# speedup vs baseline: 1.8076x; 1.8076x over previous
import jax
import jax.numpy as jnp
from jax import lax
from jax.experimental import pallas as pl
from jax.experimental.pallas import tpu as pltpu

_THRESHOLD = 0.35
_NEGPOS_RATIO = 7
_VAR0, _VAR1 = 0.1, 0.2
_B = 32
_P = 16800
_NOBJ = 32
_ROWS = 8
_COLS = 2176
_PPAD = _ROWS * _COLS

def _smooth_l1(x):
    ax = jnp.abs(x)
    return jnp.where(ax < 1.0, 0.5 * x * x, ax - 0.5)


def _lse2(a, b):
    m = jnp.maximum(a, b)
    return m + jnp.log(jnp.exp(a - m) + jnp.exp(b - m))


def _lse3(a, b, c):
    m = jnp.maximum(jnp.maximum(a, b), c)
    return m + jnp.log(jnp.exp(a - m) + jnp.exp(b - m) + jnp.exp(c - m))


def _match_body(tg_ref, priors_ref, loc_ref, conf_ref, blur_ref, expr_ref,
                illu_ref, occl_ref, pose_ref,
                out_lcp, out_l, out_cpos, out_b, out_e, out_i, out_o, out_p,
                out_np):
    f32 = jnp.float32
    bimg = pl.program_id(0)

    pcx = priors_ref[0]
    pcy = priors_ref[1]
    pw = priors_ref[2]
    ph = priors_ref[3]
    pxmin = pcx - pw / 2.0
    pymin = pcy - ph / 2.0
    pxmax = pcx + pw / 2.0
    pymax = pcy + ph / 2.0
    area_b = (pxmax - pxmin) * (pymax - pymin)

    riota = lax.broadcasted_iota(jnp.int32, (_ROWS, _COLS), 0)
    ciota = lax.broadcasted_iota(jnp.int32, (_ROWS, _COLS), 1)
    piota = riota * _COLS + ciota
    valid = piota < _P

    best_ov = jnp.full((_ROWS, _COLS), -1.0, f32)
    best_j = jnp.zeros((_ROWS, _COLS), jnp.int32)
    bp_list = []
    for j in range(_NOBJ):
        x1 = tg_ref[bimg, j, 0]
        y1 = tg_ref[bimg, j, 1]
        x2 = tg_ref[bimg, j, 2]
        y2 = tg_ref[bimg, j, 3]
        iw = jnp.maximum(jnp.minimum(x2, pxmax) - jnp.maximum(x1, pxmin), 0.0)
        ih = jnp.maximum(jnp.minimum(y2, pymax) - jnp.maximum(y1, pymin), 0.0)
        inter = iw * ih
        area_a = (x2 - x1) * (y2 - y1)
        ov = inter / (area_a + area_b - inter)
        upd = ov > best_ov
        best_ov = jnp.where(upd, ov, best_ov)
        best_j = jnp.where(upd, j, best_j)
        mx = jnp.max(ov)
        bp_list.append(jnp.min(jnp.where(ov == mx, piota, _PPAD)))
    for j in range(_NOBJ):
        m = piota == bp_list[j]
        best_ov = jnp.where(m, 2.0, best_ov)
        best_j = jnp.where(m, j, best_j)

    pos = jnp.logical_and(best_ov >= _THRESHOLD, valid)

    def gather_chan(cidx):
        acc = jnp.zeros((_ROWS, _COLS), f32)
        for j in range(_NOBJ):
            acc = jnp.where(best_j == j, tg_ref[bimg, j, cidx], acc)
        return acc

    mx1 = gather_chan(0)
    my1 = gather_chan(1)
    mx2 = gather_chan(2)
    my2 = gather_chan(3)
    ci = gather_chan(4).astype(jnp.int32)
    abl = ci & 3
    aex = (ci >> 2) & 3
    ail = (ci >> 4) & 3
    aoc = (ci >> 6) & 3
    apo = (ci >> 8) & 3

    g_cx = ((mx1 + mx2) / 2.0 - pcx) / (_VAR0 * pw)
    g_cy = ((my1 + my2) / 2.0 - pcy) / (_VAR0 * ph)
    g_w = jnp.log((mx2 - mx1) / pw) / _VAR1
    g_h = jnp.log((my2 - my1) / ph) / _VAR1
    d = loc_ref[0]
    sl = (_smooth_l1(d[0] - g_cx) + _smooth_l1(d[1] - g_cy)
          + _smooth_l1(d[2] - g_w) + _smooth_l1(d[3] - g_h))
    loss_l = jnp.sum(jnp.where(pos, sl, 0.0))

    def ce2(ref, att):
        h0 = ref[0, 0]
        h1 = ref[0, 1]
        picked = jnp.where(att == 0, h0, h1)
        return _lse2(h0, h1) - picked

    def ce3(ref, att):
        h0 = ref[0, 0]
        h1 = ref[0, 1]
        h2 = ref[0, 2]
        picked = jnp.where(att == 0, h0, jnp.where(att == 1, h1, h2))
        return _lse3(h0, h1, h2) - picked

    loss_b = jnp.sum(jnp.where(pos, ce3(blur_ref, abl), 0.0))
    loss_e = jnp.sum(jnp.where(pos, ce2(expr_ref, aex), 0.0))
    loss_i = jnp.sum(jnp.where(pos, ce2(illu_ref, ail), 0.0))
    loss_o = jnp.sum(jnp.where(pos, ce3(occl_ref, aoc), 0.0))
    loss_p = jnp.sum(jnp.where(pos, ce2(pose_ref, apo), 0.0))

    c0 = conf_ref[0, 0]
    c1 = conf_ref[0, 1]
    lse_c = _lse2(c0, c1)
    loss_c_pos = jnp.sum(jnp.where(pos, lse_c - c1, 0.0))
    lcp = jnp.where(pos, 0.0, jnp.where(valid, lse_c - c0, 0.0))
    out_lcp[0] = lcp

    num_pos_i = jnp.sum(jnp.where(pos, 1, 0))
    out_l[0, 0, 0] = loss_l
    out_cpos[0, 0, 0] = loss_c_pos
    out_b[0, 0, 0] = loss_b
    out_e[0, 0, 0] = loss_e
    out_i[0, 0, 0] = loss_i
    out_o[0, 0, 0] = loss_o
    out_p[0, 0, 0] = loss_p
    out_np[0, 0, 0] = num_pos_i.astype(f32)


def _mine_body(lcp_ref, kv_ref, out_neg):
    f32 = jnp.float32
    lcp = lcp_ref[...]
    bits = lax.bitcast_convert_type(lcp, jnp.int32)
    k = kv_ref[...]
    lo = jnp.zeros((_B, 1, 1), jnp.int32)
    hi = jnp.full((_B, 1, 1), 2147483647, jnp.int32)
    for _ in range(31):
        mid = lo + ((hi - lo) >> 1)
        cnt = jnp.sum(jnp.where(bits >= mid, 1, 0), axis=(1, 2), keepdims=True)
        ge_k = cnt >= k
        lo = jnp.where(ge_k, mid, lo)
        hi = jnp.where(ge_k, hi, mid)
    t = lax.bitcast_convert_type(lo, f32)
    gt = lcp > t
    cnt_gt = jnp.sum(jnp.where(gt, 1.0, 0.0), axis=(1, 2), keepdims=True)
    sum_gt = jnp.sum(jnp.where(gt, lcp, 0.0), axis=(1, 2), keepdims=True)
    neg_rows = sum_gt + (k.astype(f32) - cnt_gt) * t
    out_neg[0, 0] = jnp.sum(neg_rows)


def kernel(loc_data, conf_data, blur_data, expression_data, illumination_data,
           occlusion_data, pose_data, priors, targets):
    def _prep(x):
        c = x.shape[-1]
        xt = jnp.transpose(x, (0, 2, 1))
        xt = jnp.pad(xt, ((0, 0), (0, 0), (0, _PPAD - _P)))
        return xt.reshape(_B, c, _ROWS, _COLS)

    locT = _prep(loc_data)
    confT = _prep(conf_data)
    blurT = _prep(blur_data)
    exprT = _prep(expression_data)
    illuT = _prep(illumination_data)
    occlT = _prep(occlusion_data)
    poseT = _prep(pose_data)
    priorsT = jnp.pad(priors.T, ((0, 0), (0, _PPAD - _P))).reshape(
        4, _ROWS, _COLS)
    code = (targets[..., 4] + 4.0 * targets[..., 5] + 16.0 * targets[..., 6]
            + 64.0 * targets[..., 7] + 256.0 * targets[..., 8])
    tg2 = jnp.stack([targets[..., 0], targets[..., 1], targets[..., 2],
                     targets[..., 3], code], axis=-1)

    def head_spec(c):
        return pl.BlockSpec((1, c, _ROWS, _COLS), lambda b: (b, 0, 0, 0))

    outs = pl.pallas_call(
        _match_body,
        grid=(_B,),
        in_specs=[
            pl.BlockSpec(memory_space=pltpu.SMEM),
            pl.BlockSpec((4, _ROWS, _COLS), lambda b: (0, 0, 0)),
            head_spec(4), head_spec(2), head_spec(3), head_spec(2),
            head_spec(2), head_spec(3), head_spec(2),
        ],
        out_specs=[pl.BlockSpec((1, _ROWS, _COLS), lambda b: (b, 0, 0))] +
                  [pl.BlockSpec((1, 1, 1), lambda b: (b, 0, 0),
                                memory_space=pltpu.SMEM)] * 8,
        out_shape=[jax.ShapeDtypeStruct((_B, _ROWS, _COLS), jnp.float32)] +
                  [jax.ShapeDtypeStruct((_B, 1, 1), jnp.float32)] * 8,
    )(tg2, priorsT, locT, confT, blurT, exprT, illuT, occlT, poseT)

    (lcp_all, o_l, o_cpos, o_b, o_e, o_i, o_o, o_p, o_np) = outs
    kv = jnp.minimum(_NEGPOS_RATIO * o_np.astype(jnp.int32), _P - 1)

    neg_sum = pl.pallas_call(
        _mine_body,
        in_specs=[pl.BlockSpec(memory_space=pltpu.VMEM),
                  pl.BlockSpec(memory_space=pltpu.VMEM)],
        out_specs=pl.BlockSpec(memory_space=pltpu.SMEM),
        out_shape=jax.ShapeDtypeStruct((1, 1), jnp.float32),
    )(lcp_all, kv)

    s_l = jnp.sum(o_l)
    s_cpos = jnp.sum(o_cpos)
    s_b = jnp.sum(o_b)
    s_e = jnp.sum(o_e)
    s_i = jnp.sum(o_i)
    s_o = jnp.sum(o_o)
    s_p = jnp.sum(o_p)
    n = jnp.maximum(jnp.sum(o_np), 1.0)
    loss_c = s_cpos + neg_sum[0, 0]
    return (s_l / n, loss_c / n, s_b / n, s_e / n, s_i / n, s_o / n, s_p / n)


# speedup vs baseline: 1.8092x; 1.0009x over previous
import jax
import jax.numpy as jnp
from jax import lax
from jax.experimental import pallas as pl
from jax.experimental.pallas import tpu as pltpu

_THRESHOLD = 0.35
_NEGPOS_RATIO = 7
_VAR0, _VAR1 = 0.1, 0.2
_B = 32
_P = 16800
_NOBJ = 32
_ROWS = 8
_COLS = 2176
_PPAD = _ROWS * _COLS

def _smooth_l1(x):
    ax = jnp.abs(x)
    return jnp.where(ax < 1.0, 0.5 * x * x, ax - 0.5)


def _lse2(a, b):
    m = jnp.maximum(a, b)
    return m + jnp.log(jnp.exp(a - m) + jnp.exp(b - m))


def _lse3(a, b, c):
    m = jnp.maximum(jnp.maximum(a, b), c)
    return m + jnp.log(jnp.exp(a - m) + jnp.exp(b - m) + jnp.exp(c - m))


def _match_body(tg_ref, priors_ref, loc_ref, conf_ref, blur_ref, expr_ref,
                illu_ref, occl_ref, pose_ref,
                out_lcp, out_l, out_cpos, out_b, out_e, out_i, out_o, out_p,
                out_np):
    f32 = jnp.float32
    bimg = pl.program_id(0)

    pcx = priors_ref[0]
    pcy = priors_ref[1]
    pw = priors_ref[2]
    ph = priors_ref[3]
    pxmin = pcx - pw / 2.0
    pymin = pcy - ph / 2.0
    pxmax = pcx + pw / 2.0
    pymax = pcy + ph / 2.0
    area_b = (pxmax - pxmin) * (pymax - pymin)

    riota = lax.broadcasted_iota(jnp.int32, (_ROWS, _COLS), 0)
    ciota = lax.broadcasted_iota(jnp.int32, (_ROWS, _COLS), 1)
    piota = riota * _COLS + ciota
    valid = piota < _P

    best_ov = jnp.full((_ROWS, _COLS), -1.0, f32)
    best_j = jnp.zeros((_ROWS, _COLS), jnp.int32)
    bp_list = []
    for j in range(_NOBJ):
        x1 = tg_ref[bimg, j, 0]
        y1 = tg_ref[bimg, j, 1]
        x2 = tg_ref[bimg, j, 2]
        y2 = tg_ref[bimg, j, 3]
        iw = jnp.maximum(jnp.minimum(x2, pxmax) - jnp.maximum(x1, pxmin), 0.0)
        ih = jnp.maximum(jnp.minimum(y2, pymax) - jnp.maximum(y1, pymin), 0.0)
        inter = iw * ih
        area_a = (x2 - x1) * (y2 - y1)
        ov = inter / (area_a + area_b - inter)
        upd = ov > best_ov
        best_ov = jnp.where(upd, ov, best_ov)
        best_j = jnp.where(upd, j, best_j)
        mx = jnp.max(ov)
        bp_list.append(jnp.min(jnp.where(ov == mx, piota, _PPAD)))
    for j in range(_NOBJ):
        m = piota == bp_list[j]
        best_ov = jnp.where(m, 2.0, best_ov)
        best_j = jnp.where(m, j, best_j)

    pos = jnp.logical_and(best_ov >= _THRESHOLD, valid)

    def gather_chan(cidx):
        acc = jnp.zeros((_ROWS, _COLS), f32)
        for j in range(_NOBJ):
            acc = jnp.where(best_j == j, tg_ref[bimg, j, cidx], acc)
        return acc

    mx1 = gather_chan(0)
    my1 = gather_chan(1)
    mx2 = gather_chan(2)
    my2 = gather_chan(3)
    ci = gather_chan(4).astype(jnp.int32)
    abl = ci & 3
    aex = (ci >> 2) & 3
    ail = (ci >> 4) & 3
    aoc = (ci >> 6) & 3
    apo = (ci >> 8) & 3

    g_cx = ((mx1 + mx2) / 2.0 - pcx) / (_VAR0 * pw)
    g_cy = ((my1 + my2) / 2.0 - pcy) / (_VAR0 * ph)
    g_w = jnp.log((mx2 - mx1) / pw) / _VAR1
    g_h = jnp.log((my2 - my1) / ph) / _VAR1
    d = loc_ref[0]
    sl = (_smooth_l1(d[0] - g_cx) + _smooth_l1(d[1] - g_cy)
          + _smooth_l1(d[2] - g_w) + _smooth_l1(d[3] - g_h))
    loss_l = jnp.sum(jnp.where(pos, sl, 0.0))

    def ce2(ref, att):
        h0 = ref[0, 0]
        h1 = ref[0, 1]
        picked = jnp.where(att == 0, h0, h1)
        return _lse2(h0, h1) - picked

    def ce3(ref, att):
        h0 = ref[0, 0]
        h1 = ref[0, 1]
        h2 = ref[0, 2]
        picked = jnp.where(att == 0, h0, jnp.where(att == 1, h1, h2))
        return _lse3(h0, h1, h2) - picked

    loss_b = jnp.sum(jnp.where(pos, ce3(blur_ref, abl), 0.0))
    loss_e = jnp.sum(jnp.where(pos, ce2(expr_ref, aex), 0.0))
    loss_i = jnp.sum(jnp.where(pos, ce2(illu_ref, ail), 0.0))
    loss_o = jnp.sum(jnp.where(pos, ce3(occl_ref, aoc), 0.0))
    loss_p = jnp.sum(jnp.where(pos, ce2(pose_ref, apo), 0.0))

    c0 = conf_ref[0, 0]
    c1 = conf_ref[0, 1]
    lse_c = _lse2(c0, c1)
    loss_c_pos = jnp.sum(jnp.where(pos, lse_c - c1, 0.0))
    lcp = jnp.where(pos, 0.0, jnp.where(valid, lse_c - c0, 0.0))
    out_lcp[0] = lcp

    num_pos_i = jnp.sum(jnp.where(pos, 1, 0))
    out_l[0, 0, 0] = loss_l
    out_cpos[0, 0, 0] = loss_c_pos
    out_b[0, 0, 0] = loss_b
    out_e[0, 0, 0] = loss_e
    out_i[0, 0, 0] = loss_i
    out_o[0, 0, 0] = loss_o
    out_p[0, 0, 0] = loss_p
    out_np[0, 0, 0] = num_pos_i.astype(f32)


def _mine_body(lcp_ref, kv_ref, out_neg):
    f32 = jnp.float32
    lcp = lcp_ref[...]
    bits = lax.bitcast_convert_type(lcp, jnp.int32)
    k = kv_ref[...]
    lo = jnp.zeros((_B, 1, 1), jnp.int32)
    hi = jnp.full((_B, 1, 1), 2147483647, jnp.int32)
    for _ in range(31):
        mid = lo + ((hi - lo) >> 1)
        cnt = jnp.sum(jnp.where(bits >= mid, 1, 0), axis=(1, 2), keepdims=True)
        ge_k = cnt >= k
        lo = jnp.where(ge_k, mid, lo)
        hi = jnp.where(ge_k, hi, mid)
    t = lax.bitcast_convert_type(lo, f32)
    gt = lcp > t
    cnt_gt = jnp.sum(jnp.where(gt, 1.0, 0.0), axis=(1, 2), keepdims=True)
    sum_gt = jnp.sum(jnp.where(gt, lcp, 0.0), axis=(1, 2), keepdims=True)
    neg_rows = sum_gt + (k.astype(f32) - cnt_gt) * t
    out_neg[0, 0] = jnp.sum(neg_rows)


def kernel(loc_data, conf_data, blur_data, expression_data, illumination_data,
           occlusion_data, pose_data, priors, targets):
    def _prep(x):
        c = x.shape[-1]
        xt = jnp.transpose(x, (0, 2, 1))
        xt = jnp.pad(xt, ((0, 0), (0, 0), (0, _PPAD - _P)))
        return xt.reshape(_B, c, _ROWS, _COLS)

    locT = _prep(loc_data)
    confT = _prep(conf_data)
    blurT = _prep(blur_data)
    exprT = _prep(expression_data)
    illuT = _prep(illumination_data)
    occlT = _prep(occlusion_data)
    poseT = _prep(pose_data)
    priorsT = jnp.pad(priors.T, ((0, 0), (0, _PPAD - _P))).reshape(
        4, _ROWS, _COLS)
    code = (targets[..., 4] + 4.0 * targets[..., 5] + 16.0 * targets[..., 6]
            + 64.0 * targets[..., 7] + 256.0 * targets[..., 8])
    tg2 = jnp.stack([targets[..., 0], targets[..., 1], targets[..., 2],
                     targets[..., 3], code], axis=-1)

    def head_spec(c):
        return pl.BlockSpec((1, c, _ROWS, _COLS), lambda b: (b, 0, 0, 0))

    outs = pl.pallas_call(
        _match_body,
        grid=(_B,),
        in_specs=[
            pl.BlockSpec(memory_space=pltpu.SMEM),
            pl.BlockSpec((4, _ROWS, _COLS), lambda b: (0, 0, 0)),
            head_spec(4), head_spec(2), head_spec(3), head_spec(2),
            head_spec(2), head_spec(3), head_spec(2),
        ],
        out_specs=[pl.BlockSpec((1, _ROWS, _COLS), lambda b: (b, 0, 0))] +
                  [pl.BlockSpec((1, 1, 1), lambda b: (b, 0, 0),
                                memory_space=pltpu.SMEM)] * 8,
        out_shape=[jax.ShapeDtypeStruct((_B, _ROWS, _COLS), jnp.float32)] +
                  [jax.ShapeDtypeStruct((_B, 1, 1), jnp.float32)] * 8,
        compiler_params=pltpu.CompilerParams(
            dimension_semantics=("parallel",)),
    )(tg2, priorsT, locT, confT, blurT, exprT, illuT, occlT, poseT)

    (lcp_all, o_l, o_cpos, o_b, o_e, o_i, o_o, o_p, o_np) = outs
    kv = jnp.minimum(_NEGPOS_RATIO * o_np.astype(jnp.int32), _P - 1)

    neg_sum = pl.pallas_call(
        _mine_body,
        in_specs=[pl.BlockSpec(memory_space=pltpu.VMEM),
                  pl.BlockSpec(memory_space=pltpu.VMEM)],
        out_specs=pl.BlockSpec(memory_space=pltpu.SMEM),
        out_shape=jax.ShapeDtypeStruct((1, 1), jnp.float32),
    )(lcp_all, kv)

    s_l = jnp.sum(o_l)
    s_cpos = jnp.sum(o_cpos)
    s_b = jnp.sum(o_b)
    s_e = jnp.sum(o_e)
    s_i = jnp.sum(o_i)
    s_o = jnp.sum(o_o)
    s_p = jnp.sum(o_p)
    n = jnp.maximum(jnp.sum(o_np), 1.0)
    loss_c = s_cpos + neg_sum[0, 0]
    return (s_l / n, loss_c / n, s_b / n, s_e / n, s_i / n, s_o / n, s_p / n)


# speedup vs baseline: 1.8330x; 1.0131x over previous
import jax
import jax.numpy as jnp
from jax import lax
from jax.experimental import pallas as pl
from jax.experimental.pallas import tpu as pltpu

_THRESHOLD = 0.35
_NEGPOS_RATIO = 7
_VAR0, _VAR1 = 0.1, 0.2
_B = 32
_P = 16800
_NOBJ = 32
_ROWS = 8
_COLS = 2176
_PPAD = _ROWS * _COLS

def _smooth_l1(x):
    ax = jnp.abs(x)
    return jnp.where(ax < 1.0, 0.5 * x * x, ax - 0.5)


def _lse2(a, b):
    m = jnp.maximum(a, b)
    return m + jnp.log(jnp.exp(a - m) + jnp.exp(b - m))


def _lse3(a, b, c):
    m = jnp.maximum(jnp.maximum(a, b), c)
    return m + jnp.log(jnp.exp(a - m) + jnp.exp(b - m) + jnp.exp(c - m))


def _match_body(tg_ref, priors_ref, loc_ref, conf_ref, blur_ref, expr_ref,
                illu_ref, occl_ref, pose_ref,
                out_lcp, out_l, out_cpos, out_b, out_e, out_i, out_o, out_p,
                out_np):
    f32 = jnp.float32
    bimg = pl.program_id(0)

    pcx = priors_ref[0]
    pcy = priors_ref[1]
    pw = priors_ref[2]
    ph = priors_ref[3]
    pxmin = pcx - pw / 2.0
    pymin = pcy - ph / 2.0
    pxmax = pcx + pw / 2.0
    pymax = pcy + ph / 2.0
    area_b = (pxmax - pxmin) * (pymax - pymin)

    riota = lax.broadcasted_iota(jnp.int32, (_ROWS, _COLS), 0)
    ciota = lax.broadcasted_iota(jnp.int32, (_ROWS, _COLS), 1)
    piota = riota * _COLS + ciota
    valid = piota < _P

    best_ov = jnp.full((_ROWS, _COLS), -1.0, f32)
    macc = [jnp.zeros((_ROWS, _COLS), f32) for _ in range(5)]
    bp_list = []
    for j in range(_NOBJ):
        x1 = tg_ref[bimg, j, 0]
        y1 = tg_ref[bimg, j, 1]
        x2 = tg_ref[bimg, j, 2]
        y2 = tg_ref[bimg, j, 3]
        iw = jnp.maximum(jnp.minimum(x2, pxmax) - jnp.maximum(x1, pxmin), 0.0)
        ih = jnp.maximum(jnp.minimum(y2, pymax) - jnp.maximum(y1, pymin), 0.0)
        inter = iw * ih
        area_a = (x2 - x1) * (y2 - y1)
        ov = inter / (area_a + area_b - inter)
        upd = ov > best_ov
        best_ov = jnp.where(upd, ov, best_ov)
        for c in range(5):
            macc[c] = jnp.where(upd, tg_ref[bimg, j, c], macc[c])
        mx = jnp.max(ov)
        bp_list.append(jnp.min(jnp.where(ov == mx, piota, _PPAD)))
    for j in range(_NOBJ):
        m = piota == bp_list[j]
        best_ov = jnp.where(m, 2.0, best_ov)
        for c in range(5):
            macc[c] = jnp.where(m, tg_ref[bimg, j, c], macc[c])

    pos = jnp.logical_and(best_ov >= _THRESHOLD, valid)

    mx1, my1, mx2, my2 = macc[0], macc[1], macc[2], macc[3]
    ci = macc[4].astype(jnp.int32)
    abl = ci & 3
    aex = (ci >> 2) & 3
    ail = (ci >> 4) & 3
    aoc = (ci >> 6) & 3
    apo = (ci >> 8) & 3

    g_cx = ((mx1 + mx2) / 2.0 - pcx) / (_VAR0 * pw)
    g_cy = ((my1 + my2) / 2.0 - pcy) / (_VAR0 * ph)
    g_w = jnp.log((mx2 - mx1) / pw) / _VAR1
    g_h = jnp.log((my2 - my1) / ph) / _VAR1
    d = loc_ref[0]
    sl = (_smooth_l1(d[0] - g_cx) + _smooth_l1(d[1] - g_cy)
          + _smooth_l1(d[2] - g_w) + _smooth_l1(d[3] - g_h))
    loss_l = jnp.sum(jnp.where(pos, sl, 0.0))

    def ce2(ref, att):
        h0 = ref[0, 0]
        h1 = ref[0, 1]
        picked = jnp.where(att == 0, h0, h1)
        return _lse2(h0, h1) - picked

    def ce3(ref, att):
        h0 = ref[0, 0]
        h1 = ref[0, 1]
        h2 = ref[0, 2]
        picked = jnp.where(att == 0, h0, jnp.where(att == 1, h1, h2))
        return _lse3(h0, h1, h2) - picked

    loss_b = jnp.sum(jnp.where(pos, ce3(blur_ref, abl), 0.0))
    loss_e = jnp.sum(jnp.where(pos, ce2(expr_ref, aex), 0.0))
    loss_i = jnp.sum(jnp.where(pos, ce2(illu_ref, ail), 0.0))
    loss_o = jnp.sum(jnp.where(pos, ce3(occl_ref, aoc), 0.0))
    loss_p = jnp.sum(jnp.where(pos, ce2(pose_ref, apo), 0.0))

    c0 = conf_ref[0, 0]
    c1 = conf_ref[0, 1]
    lse_c = _lse2(c0, c1)
    loss_c_pos = jnp.sum(jnp.where(pos, lse_c - c1, 0.0))
    lcp = jnp.where(pos, 0.0, jnp.where(valid, lse_c - c0, 0.0))
    out_lcp[0] = lcp

    num_pos_i = jnp.sum(jnp.where(pos, 1, 0))
    out_l[0, 0, 0] = loss_l
    out_cpos[0, 0, 0] = loss_c_pos
    out_b[0, 0, 0] = loss_b
    out_e[0, 0, 0] = loss_e
    out_i[0, 0, 0] = loss_i
    out_o[0, 0, 0] = loss_o
    out_p[0, 0, 0] = loss_p
    out_np[0, 0, 0] = num_pos_i.astype(f32)


def _mine_body(lcp_ref, kv_ref, out_neg):
    f32 = jnp.float32
    lcp = lcp_ref[...]
    bits = lax.bitcast_convert_type(lcp, jnp.int32)
    k = kv_ref[...]
    lo = jnp.zeros((_B, 1, 1), jnp.int32)
    hi = jnp.full((_B, 1, 1), 2147483647, jnp.int32)
    for _ in range(31):
        mid = lo + ((hi - lo) >> 1)
        cnt = jnp.sum(jnp.where(bits >= mid, 1, 0), axis=(1, 2), keepdims=True)
        ge_k = cnt >= k
        lo = jnp.where(ge_k, mid, lo)
        hi = jnp.where(ge_k, hi, mid)
    t = lax.bitcast_convert_type(lo, f32)
    gt = lcp > t
    cnt_gt = jnp.sum(jnp.where(gt, 1.0, 0.0), axis=(1, 2), keepdims=True)
    sum_gt = jnp.sum(jnp.where(gt, lcp, 0.0), axis=(1, 2), keepdims=True)
    neg_rows = sum_gt + (k.astype(f32) - cnt_gt) * t
    out_neg[0, 0] = jnp.sum(neg_rows)


def kernel(loc_data, conf_data, blur_data, expression_data, illumination_data,
           occlusion_data, pose_data, priors, targets):
    def _prep(x):
        c = x.shape[-1]
        xt = jnp.transpose(x, (0, 2, 1))
        xt = jnp.pad(xt, ((0, 0), (0, 0), (0, _PPAD - _P)))
        return xt.reshape(_B, c, _ROWS, _COLS)

    locT = _prep(loc_data)
    confT = _prep(conf_data)
    blurT = _prep(blur_data)
    exprT = _prep(expression_data)
    illuT = _prep(illumination_data)
    occlT = _prep(occlusion_data)
    poseT = _prep(pose_data)
    priorsT = jnp.pad(priors.T, ((0, 0), (0, _PPAD - _P))).reshape(
        4, _ROWS, _COLS)
    code = (targets[..., 4] + 4.0 * targets[..., 5] + 16.0 * targets[..., 6]
            + 64.0 * targets[..., 7] + 256.0 * targets[..., 8])
    tg2 = jnp.stack([targets[..., 0], targets[..., 1], targets[..., 2],
                     targets[..., 3], code], axis=-1)

    def head_spec(c):
        return pl.BlockSpec((1, c, _ROWS, _COLS), lambda b: (b, 0, 0, 0))

    outs = pl.pallas_call(
        _match_body,
        grid=(_B,),
        in_specs=[
            pl.BlockSpec(memory_space=pltpu.SMEM),
            pl.BlockSpec((4, _ROWS, _COLS), lambda b: (0, 0, 0)),
            head_spec(4), head_spec(2), head_spec(3), head_spec(2),
            head_spec(2), head_spec(3), head_spec(2),
        ],
        out_specs=[pl.BlockSpec((1, _ROWS, _COLS), lambda b: (b, 0, 0))] +
                  [pl.BlockSpec((1, 1, 1), lambda b: (b, 0, 0),
                                memory_space=pltpu.SMEM)] * 8,
        out_shape=[jax.ShapeDtypeStruct((_B, _ROWS, _COLS), jnp.float32)] +
                  [jax.ShapeDtypeStruct((_B, 1, 1), jnp.float32)] * 8,
        compiler_params=pltpu.CompilerParams(
            dimension_semantics=("parallel",)),
    )(tg2, priorsT, locT, confT, blurT, exprT, illuT, occlT, poseT)

    (lcp_all, o_l, o_cpos, o_b, o_e, o_i, o_o, o_p, o_np) = outs
    kv = jnp.minimum(_NEGPOS_RATIO * o_np.astype(jnp.int32), _P - 1)

    neg_sum = pl.pallas_call(
        _mine_body,
        in_specs=[pl.BlockSpec(memory_space=pltpu.VMEM),
                  pl.BlockSpec(memory_space=pltpu.VMEM)],
        out_specs=pl.BlockSpec(memory_space=pltpu.SMEM),
        out_shape=jax.ShapeDtypeStruct((1, 1), jnp.float32),
    )(lcp_all, kv)

    s_l = jnp.sum(o_l)
    s_cpos = jnp.sum(o_cpos)
    s_b = jnp.sum(o_b)
    s_e = jnp.sum(o_e)
    s_i = jnp.sum(o_i)
    s_o = jnp.sum(o_o)
    s_p = jnp.sum(o_p)
    n = jnp.maximum(jnp.sum(o_np), 1.0)
    loss_c = s_cpos + neg_sum[0, 0]
    return (s_l / n, loss_c / n, s_b / n, s_e / n, s_i / n, s_o / n, s_p / n)
